# baseline (speedup 1.0000x reference)
import functools

import jax
import jax.numpy as jnp
from jax import lax
from jax.experimental import pallas as pl
from jax.experimental.pallas import tpu as pltpu
from jax.experimental.pallas import tpu_sc as plsc

_MIN_VAL = -20.0
_MAX_VAL = 20.0
_BINS = 255
_STEP = (_MAX_VAL - _MIN_VAL) / (_BINS - 1)

_N = 262144
_NC = 2
_NS = 16
_NW = _NC * _NS
_CHUNK = _N // _NW
_R = 128
_NB = _CHUNK // _R
_BLK = _R * _BINS
_NBUF = 2
_LANES = 16
_GROUPS = _R // _LANES


def _group_indices(xv, g, j):
    xg = xv[pl.ds(g * _R + j * _LANES, _LANES)]
    xc = jnp.minimum(jnp.maximum(xg, _MIN_VAL), _MAX_VAL)
    p = (xc - _MIN_VAL) / _STEP
    li = p.astype(jnp.int32)
    uw = p - li.astype(jnp.float32)
    lw = 1.0 - uw
    ui = jnp.minimum(li + 1, _BINS - 1)
    flat = (lax.iota(jnp.int32, _LANES) + j * _LANES) * _BINS
    return flat + li, flat + ui, lw, uw


def _sc_body(x_hbm, out_hbm, xv, bufs, sems):
    wid = lax.axis_index("s") * _NC + lax.axis_index("c")
    base_row = wid * _CHUNK
    pltpu.sync_copy(x_hbm.at[pl.ds(base_row, _CHUNK)], xv)

    def _zero_init(i, carry):
        z = jnp.zeros((_LANES,), jnp.float32)
        for buf in bufs:
            buf[pl.ds(i * _LANES, _LANES)] = z
        return carry

    lax.fori_loop(0, _BLK // _LANES, _zero_init, 0)

    def _write_block(buf, g):
        for j in range(_GROUPS):
            flo, fhi, lw, uw = _group_indices(xv, g, j)
            plsc.store_scatter(buf, (flo,), lw)
            plsc.addupdate_scatter(buf, (fhi,), uw)

    def _clear_block(buf, g):
        z = jnp.zeros((_LANES,), jnp.float32)
        for j in range(_GROUPS):
            flo, fhi, _, _ = _group_indices(xv, g, j)
            plsc.store_scatter(buf, (flo,), z)
            plsc.store_scatter(buf, (fhi,), z)

    def _dst(g):
        return out_hbm.at[pl.ds((base_row + g * _R) * _BINS, _BLK)]

    def _loop(i, carry):
        for b in range(_NBUF):
            g = i * _NBUF + b

            @pl.when(i > 0)
            def _():
                pltpu.make_async_copy(bufs[b], _dst(g - _NBUF), sems[b]).wait()
                _clear_block(bufs[b], g - _NBUF)

            _write_block(bufs[b], g)
            pltpu.async_copy(bufs[b], _dst(g), sems[b])
        return carry

    lax.fori_loop(0, _NB // _NBUF, _loop, 0)
    for b in range(_NBUF):
        pltpu.make_async_copy(bufs[b], _dst(_NB - _NBUF + b), sems[b]).wait()


@functools.partial(
    pl.kernel,
    out_type=jax.ShapeDtypeStruct((_N * _BINS,), jnp.float32),
    mesh=plsc.VectorSubcoreMesh(
        core_axis_name="c", subcore_axis_name="s", num_cores=_NC,
        num_subcores=_NS),
    scratch_types=[
        pltpu.VMEM((_CHUNK,), jnp.float32),
        [pltpu.VMEM((_BLK,), jnp.float32) for _ in range(_NBUF)],
        [pltpu.SemaphoreType.DMA for _ in range(_NBUF)],
    ],
    compiler_params=pltpu.CompilerParams(needs_layout_passes=False),
)
def _two_hot_sc(x_hbm, out_hbm, xv, bufs, sems):
    _sc_body(x_hbm, out_hbm, xv, tuple(bufs), tuple(sems))


def kernel(x):
    flat = _two_hot_sc(x.reshape(-1))
    return flat.reshape(_N, _BINS)


# speedup vs baseline: 4.7239x; 4.7239x over previous
import functools

import jax
import jax.numpy as jnp
from jax import lax
from jax.experimental import pallas as pl
from jax.experimental.pallas import tpu as pltpu
from jax.experimental.pallas import tpu_sc as plsc

_MIN_VAL = -20.0
_MAX_VAL = 20.0
_BINS = 255
_STEP = (_MAX_VAL - _MIN_VAL) / (_BINS - 1)

_N = 262144
_NC = 2
_NS = 16
_NW = _NC * _NS
_CHUNK = _N // _NW
_R = 128
_NB = _CHUNK // _R
_NBUF = 2
_LANES = 16
_GROUPS = _R // _LANES


def _group_indices(xv, g, j):
    xg = xv[pl.ds(g * _R + j * _LANES, _LANES)]
    xc = jnp.minimum(jnp.maximum(xg, _MIN_VAL), _MAX_VAL)
    p = (xc - _MIN_VAL) / _STEP
    li = p.astype(jnp.int32)
    uw = p - li.astype(jnp.float32)
    lw = 1.0 - uw
    ui = jnp.minimum(li + 1, _BINS - 1)
    rows = lax.iota(jnp.int32, _LANES) + j * _LANES
    return rows, li, ui, lw, uw


def _sc_body(x_hbm, out_hbm, xv, bufs, sems):
    wid = lax.axis_index("s") * _NC + lax.axis_index("c")
    base_row = wid * _CHUNK
    pltpu.sync_copy(x_hbm.at[pl.ds(base_row, _CHUNK)], xv)

    def _zero_init(i, carry):
        z = jnp.zeros((_LANES,), jnp.float32)
        r = i // (_BINS // _LANES + 1)
        c = (i % (_BINS // _LANES + 1)) * _LANES
        c = jnp.minimum(c, _BINS - _LANES)
        for buf in bufs:
            rows = jnp.full((_LANES,), r, jnp.int32)
            cols = c + lax.iota(jnp.int32, _LANES)
            plsc.store_scatter(buf, (rows, cols), z)
        return carry

    lax.fori_loop(0, _R * (_BINS // _LANES + 1), _zero_init, 0)

    def _write_block(buf, g):
        for j in range(_GROUPS):
            rows, li, ui, lw, uw = _group_indices(xv, g, j)
            plsc.store_scatter(buf, (rows, li), lw)
            plsc.addupdate_scatter(buf, (rows, ui), uw)

    def _clear_block(buf, g):
        z = jnp.zeros((_LANES,), jnp.float32)
        for j in range(_GROUPS):
            rows, li, ui, _, _ = _group_indices(xv, g, j)
            plsc.store_scatter(buf, (rows, li), z)
            plsc.store_scatter(buf, (rows, ui), z)

    def _dst(g):
        return out_hbm.at[pl.ds(base_row + g * _R, _R)]

    def _loop(i, carry):
        for b in range(_NBUF):
            g = i * _NBUF + b

            @pl.when(i > 0)
            def _():
                pltpu.make_async_copy(bufs[b], _dst(g - _NBUF), sems[b]).wait()
                _clear_block(bufs[b], g - _NBUF)

            _write_block(bufs[b], g)
            pltpu.async_copy(bufs[b], _dst(g), sems[b])
        return carry

    lax.fori_loop(0, _NB // _NBUF, _loop, 0)
    for b in range(_NBUF):
        pltpu.make_async_copy(bufs[b], _dst(_NB - _NBUF + b), sems[b]).wait()


@functools.partial(
    pl.kernel,
    out_type=jax.ShapeDtypeStruct((_N, _BINS), jnp.float32),
    mesh=plsc.VectorSubcoreMesh(
        core_axis_name="c", subcore_axis_name="s", num_cores=_NC,
        num_subcores=_NS),
    scratch_types=[
        pltpu.VMEM((_CHUNK,), jnp.float32),
        [pltpu.VMEM((_R, _BINS), jnp.float32) for _ in range(_NBUF)],
        [pltpu.SemaphoreType.DMA for _ in range(_NBUF)],
    ],
    compiler_params=pltpu.CompilerParams(needs_layout_passes=False),
)
def _two_hot_sc(x_hbm, out_hbm, xv, bufs, sems):
    _sc_body(x_hbm, out_hbm, xv, tuple(bufs), tuple(sems))


def kernel(x):
    return _two_hot_sc(x.reshape(-1))


# speedup vs baseline: 4.8415x; 1.0249x over previous
import functools

import jax
import jax.numpy as jnp
from jax import lax
from jax.experimental import pallas as pl
from jax.experimental.pallas import tpu as pltpu
from jax.experimental.pallas import tpu_sc as plsc

_MIN_VAL = -20.0
_MAX_VAL = 20.0
_BINS = 255
_STEP = (_MAX_VAL - _MIN_VAL) / (_BINS - 1)

_N = 262144
_NC = 2
_NS = 16
_NW = _NC * _NS
_CHUNK = _N // _NW
_R = 64
_NB = _CHUNK // _R
_NBUF = 4
_LANES = 16
_GROUPS = _R // _LANES


def _group_indices(xv, g, j):
    xg = xv[pl.ds(g * _R + j * _LANES, _LANES)]
    xc = jnp.minimum(jnp.maximum(xg, _MIN_VAL), _MAX_VAL)
    p = (xc - _MIN_VAL) / _STEP
    li = p.astype(jnp.int32)
    uw = p - li.astype(jnp.float32)
    lw = 1.0 - uw
    ui = jnp.minimum(li + 1, _BINS - 1)
    rows = lax.iota(jnp.int32, _LANES) + j * _LANES
    return rows, li, ui, lw, uw


def _sc_body(x_hbm, out_hbm, xv, bufs, sems):
    wid = lax.axis_index("s") * _NC + lax.axis_index("c")
    base_row = wid * _CHUNK
    pltpu.sync_copy(x_hbm.at[pl.ds(base_row, _CHUNK)], xv)

    def _zero_init(i, carry):
        z = jnp.zeros((_LANES,), jnp.float32)
        r = i // (_BINS // _LANES + 1)
        c = (i % (_BINS // _LANES + 1)) * _LANES
        c = jnp.minimum(c, _BINS - _LANES)
        for buf in bufs:
            rows = jnp.full((_LANES,), r, jnp.int32)
            cols = c + lax.iota(jnp.int32, _LANES)
            plsc.store_scatter(buf, (rows, cols), z)
        return carry

    lax.fori_loop(0, _R * (_BINS // _LANES + 1), _zero_init, 0)

    def _write_block(buf, g):
        for j in range(_GROUPS):
            rows, li, ui, lw, uw = _group_indices(xv, g, j)
            plsc.store_scatter(buf, (rows, li), lw)
            plsc.addupdate_scatter(buf, (rows, ui), uw)

    def _clear_block(buf, g):
        z = jnp.zeros((_LANES,), jnp.float32)
        for j in range(_GROUPS):
            rows, li, ui, _, _ = _group_indices(xv, g, j)
            plsc.store_scatter(buf, (rows, li), z)
            plsc.store_scatter(buf, (rows, ui), z)

    def _dst(g):
        return out_hbm.at[pl.ds(base_row + g * _R, _R)]

    def _loop(i, carry):
        for b in range(_NBUF):
            g = i * _NBUF + b

            @pl.when(i > 0)
            def _():
                pltpu.make_async_copy(bufs[b], _dst(g - _NBUF), sems[b]).wait()
                _clear_block(bufs[b], g - _NBUF)

            _write_block(bufs[b], g)
            pltpu.async_copy(bufs[b], _dst(g), sems[b])
        return carry

    lax.fori_loop(0, _NB // _NBUF, _loop, 0)
    for b in range(_NBUF):
        pltpu.make_async_copy(bufs[b], _dst(_NB - _NBUF + b), sems[b]).wait()


@functools.partial(
    pl.kernel,
    out_type=jax.ShapeDtypeStruct((_N, _BINS), jnp.float32),
    mesh=plsc.VectorSubcoreMesh(
        core_axis_name="c", subcore_axis_name="s", num_cores=_NC,
        num_subcores=_NS),
    scratch_types=[
        pltpu.VMEM((_CHUNK,), jnp.float32),
        [pltpu.VMEM((_R, _BINS), jnp.float32) for _ in range(_NBUF)],
        [pltpu.SemaphoreType.DMA for _ in range(_NBUF)],
    ],
    compiler_params=pltpu.CompilerParams(needs_layout_passes=False),
)
def _two_hot_sc(x_hbm, out_hbm, xv, bufs, sems):
    _sc_body(x_hbm, out_hbm, xv, tuple(bufs), tuple(sems))


def kernel(x):
    return _two_hot_sc(x.reshape(-1))


# speedup vs baseline: 4.9172x; 1.0156x over previous
import functools

import jax
import jax.numpy as jnp
from jax import lax
from jax.experimental import pallas as pl
from jax.experimental.pallas import tpu as pltpu
from jax.experimental.pallas import tpu_sc as plsc

_MIN_VAL = -20.0
_MAX_VAL = 20.0
_BINS = 255
_STEP = (_MAX_VAL - _MIN_VAL) / (_BINS - 1)

_N = 262144
_NC = 2
_NS = 16
_NW = _NC * _NS
_CHUNK = _N // _NW
_R = 32
_NB = _CHUNK // _R
_NBUF = 8
_LANES = 16
_GROUPS = _R // _LANES


def _group_indices(xv, g, j):
    xg = xv[pl.ds(g * _R + j * _LANES, _LANES)]
    xc = jnp.minimum(jnp.maximum(xg, _MIN_VAL), _MAX_VAL)
    p = (xc - _MIN_VAL) / _STEP
    li = p.astype(jnp.int32)
    uw = p - li.astype(jnp.float32)
    lw = 1.0 - uw
    ui = jnp.minimum(li + 1, _BINS - 1)
    rows = lax.iota(jnp.int32, _LANES) + j * _LANES
    return rows, li, ui, lw, uw


def _sc_body(x_hbm, out_hbm, xv, bufs, sems):
    wid = lax.axis_index("s") * _NC + lax.axis_index("c")
    base_row = wid * _CHUNK
    pltpu.sync_copy(x_hbm.at[pl.ds(base_row, _CHUNK)], xv)

    def _zero_init(i, carry):
        z = jnp.zeros((_LANES,), jnp.float32)
        r = i // (_BINS // _LANES + 1)
        c = (i % (_BINS // _LANES + 1)) * _LANES
        c = jnp.minimum(c, _BINS - _LANES)
        for buf in bufs:
            rows = jnp.full((_LANES,), r, jnp.int32)
            cols = c + lax.iota(jnp.int32, _LANES)
            plsc.store_scatter(buf, (rows, cols), z)
        return carry

    lax.fori_loop(0, _R * (_BINS // _LANES + 1), _zero_init, 0)

    def _write_block(buf, g):
        for j in range(_GROUPS):
            rows, li, ui, lw, uw = _group_indices(xv, g, j)
            plsc.store_scatter(buf, (rows, li), lw)
            plsc.addupdate_scatter(buf, (rows, ui), uw)

    def _clear_block(buf, g):
        z = jnp.zeros((_LANES,), jnp.float32)
        for j in range(_GROUPS):
            rows, li, ui, _, _ = _group_indices(xv, g, j)
            plsc.store_scatter(buf, (rows, li), z)
            plsc.store_scatter(buf, (rows, ui), z)

    def _dst(g):
        return out_hbm.at[pl.ds(base_row + g * _R, _R)]

    def _loop(i, carry):
        for b in range(_NBUF):
            g = i * _NBUF + b

            @pl.when(i > 0)
            def _():
                pltpu.make_async_copy(bufs[b], _dst(g - _NBUF), sems[b]).wait()
                _clear_block(bufs[b], g - _NBUF)

            _write_block(bufs[b], g)
            pltpu.async_copy(bufs[b], _dst(g), sems[b])
        return carry

    lax.fori_loop(0, _NB // _NBUF, _loop, 0)
    for b in range(_NBUF):
        pltpu.make_async_copy(bufs[b], _dst(_NB - _NBUF + b), sems[b]).wait()


@functools.partial(
    pl.kernel,
    out_type=jax.ShapeDtypeStruct((_N, _BINS), jnp.float32),
    mesh=plsc.VectorSubcoreMesh(
        core_axis_name="c", subcore_axis_name="s", num_cores=_NC,
        num_subcores=_NS),
    scratch_types=[
        pltpu.VMEM((_CHUNK,), jnp.float32),
        [pltpu.VMEM((_R, _BINS), jnp.float32) for _ in range(_NBUF)],
        [pltpu.SemaphoreType.DMA for _ in range(_NBUF)],
    ],
    compiler_params=pltpu.CompilerParams(needs_layout_passes=False),
)
def _two_hot_sc(x_hbm, out_hbm, xv, bufs, sems):
    _sc_body(x_hbm, out_hbm, xv, tuple(bufs), tuple(sems))


def kernel(x):
    return _two_hot_sc(x.reshape(-1))


# speedup vs baseline: 4.9410x; 1.0049x over previous
import functools

import jax
import jax.numpy as jnp
from jax import lax
from jax.experimental import pallas as pl
from jax.experimental.pallas import tpu as pltpu
from jax.experimental.pallas import tpu_sc as plsc

_MIN_VAL = -20.0
_MAX_VAL = 20.0
_BINS = 255
_STEP = (_MAX_VAL - _MIN_VAL) / (_BINS - 1)

_N = 262144
_NC = 2
_NS = 16
_NW = _NC * _NS
_CHUNK = _N // _NW
_R = 16
_NB = _CHUNK // _R
_NBUF = 16
_LANES = 16
_GROUPS = _R // _LANES


def _group_indices(xv, g, j):
    xg = xv[pl.ds(g * _R + j * _LANES, _LANES)]
    xc = jnp.minimum(jnp.maximum(xg, _MIN_VAL), _MAX_VAL)
    p = (xc - _MIN_VAL) / _STEP
    li = p.astype(jnp.int32)
    uw = p - li.astype(jnp.float32)
    lw = 1.0 - uw
    ui = jnp.minimum(li + 1, _BINS - 1)
    rows = lax.iota(jnp.int32, _LANES) + j * _LANES
    return rows, li, ui, lw, uw


def _sc_body(x_hbm, out_hbm, xv, bufs, sems):
    wid = lax.axis_index("s") * _NC + lax.axis_index("c")
    base_row = wid * _CHUNK
    pltpu.sync_copy(x_hbm.at[pl.ds(base_row, _CHUNK)], xv)

    def _zero_init(i, carry):
        z = jnp.zeros((_LANES,), jnp.float32)
        r = i // (_BINS // _LANES + 1)
        c = (i % (_BINS // _LANES + 1)) * _LANES
        c = jnp.minimum(c, _BINS - _LANES)
        for buf in bufs:
            rows = jnp.full((_LANES,), r, jnp.int32)
            cols = c + lax.iota(jnp.int32, _LANES)
            plsc.store_scatter(buf, (rows, cols), z)
        return carry

    lax.fori_loop(0, _R * (_BINS // _LANES + 1), _zero_init, 0)

    def _write_block(buf, g):
        for j in range(_GROUPS):
            rows, li, ui, lw, uw = _group_indices(xv, g, j)
            plsc.store_scatter(buf, (rows, li), lw)
            plsc.addupdate_scatter(buf, (rows, ui), uw)

    def _clear_block(buf, g):
        z = jnp.zeros((_LANES,), jnp.float32)
        for j in range(_GROUPS):
            rows, li, ui, _, _ = _group_indices(xv, g, j)
            plsc.store_scatter(buf, (rows, li), z)
            plsc.store_scatter(buf, (rows, ui), z)

    def _dst(g):
        return out_hbm.at[pl.ds(base_row + g * _R, _R)]

    def _loop(i, carry):
        for b in range(_NBUF):
            g = i * _NBUF + b

            @pl.when(i > 0)
            def _():
                pltpu.make_async_copy(bufs[b], _dst(g - _NBUF), sems[b]).wait()
                _clear_block(bufs[b], g - _NBUF)

            _write_block(bufs[b], g)
            pltpu.async_copy(bufs[b], _dst(g), sems[b])
        return carry

    lax.fori_loop(0, _NB // _NBUF, _loop, 0)
    for b in range(_NBUF):
        pltpu.make_async_copy(bufs[b], _dst(_NB - _NBUF + b), sems[b]).wait()


@functools.partial(
    pl.kernel,
    out_type=jax.ShapeDtypeStruct((_N, _BINS), jnp.float32),
    mesh=plsc.VectorSubcoreMesh(
        core_axis_name="c", subcore_axis_name="s", num_cores=_NC,
        num_subcores=_NS),
    scratch_types=[
        pltpu.VMEM((_CHUNK,), jnp.float32),
        [pltpu.VMEM((_R, _BINS), jnp.float32) for _ in range(_NBUF)],
        [pltpu.SemaphoreType.DMA for _ in range(_NBUF)],
    ],
    compiler_params=pltpu.CompilerParams(needs_layout_passes=False),
)
def _two_hot_sc(x_hbm, out_hbm, xv, bufs, sems):
    _sc_body(x_hbm, out_hbm, xv, tuple(bufs), tuple(sems))


def kernel(x):
    return _two_hot_sc(x.reshape(-1))
